# baseline (speedup 1.0000x reference)
import functools

import jax
import jax.numpy as jnp
from jax import lax
from jax.experimental import pallas as pl
from jax.experimental.pallas import tpu as pltpu
from jax.experimental.pallas import tpu_sc as plsc

DIM = 64
G = 16
NB = 8
ROWS = 4

_info = plsc.get_sparse_core_info()
NC, NS = _info.num_cores, _info.num_subcores
NW = NC * NS


def _build(b, s):
    tpw = b * s // NW
    assert tpw % 128 == 0 and tpw % (ROWS * s) == 0
    trows = tpw // 128
    nch = tpw // G
    nob = tpw // (ROWS * s)
    mesh = plsc.VectorSubcoreMesh(core_axis_name="c", subcore_axis_name="s")

    @functools.partial(
        pl.kernel,
        mesh=mesh,
        out_type=jax.ShapeDtypeStruct((b, s, DIM), jnp.float32),
        scratch_types=[
            pltpu.VMEM((trows, 128), jnp.int32),
            pltpu.VMEM((trows, 128), jnp.int32),
            pltpu.VMEM((NB, G, 128), jnp.float32),
            pltpu.VMEM((2, ROWS, s, DIM), jnp.float32),
            pltpu.SemaphoreType.DMA((NB,)),
            pltpu.SemaphoreType.DMA((2,)),
        ],
        compiler_params=pltpu.CompilerParams(use_tc_tiling_on_sc=True),
    )
    def k(tok_hbm, table_hbm, out_hbm, hvv, tidv, tiles_v, obuf, gsem, ssem):
        wid = lax.axis_index("s") * NC + lax.axis_index("c")
        row_base = wid * (tpw // s)
        pltpu.sync_copy(tok_hbm.at[wid], hvv)

        def tid_body(kk, carry):
            r, o = carry
            t16 = hvv[r, pl.ds(o, 16)]
            tidv[r, pl.ds(o, 16)] = lax.shift_right_logical(t16, 1)
            hvv[r, pl.ds(o, 16)] = lax.shift_left(lax.bitwise_and(t16, 1), 6)
            wrap = o == 112
            return (r + wrap.astype(jnp.int32),
                    lax.select(wrap, jnp.int32(0), o + 16))

        lax.fori_loop(0, trows * 8, tid_body, (jnp.int32(0), jnp.int32(0)))

        def idx_slice(r, o):
            return tidv.at[r, pl.ds(o, G)]

        for p in range(NB):
            pltpu.async_copy(
                table_hbm.at[idx_slice((p * G) // 128, (p * G) % 128)],
                tiles_v.at[p], gsem.at[p])

        def store_block(pm, m):
            pltpu.async_copy(obuf.at[pm],
                             out_hbm.at[pl.ds(row_base + m * ROWS, ROWS)],
                             ssem.at[pm])

        def wait_store(pm, m):
            pltpu.make_async_copy(obuf.at[pm],
                                  out_hbm.at[pl.ds(row_base + m * ROWS, ROWS)],
                                  ssem.at[pm]).wait()

        def step(c, carry):
            ring, r, o, pm, m, a, bb = carry

            pltpu.make_async_copy(table_hbm.at[idx_slice(r, o)],
                                  tiles_v.at[ring], gsem.at[ring]).wait()

            hv = hvv[r, pl.ds(o, 16)]
            for l in range(G):
                rr = hv[l]
                for v in range(DIM // 16):
                    obuf[pm, a, bb, pl.ds(16 * v, 16)] = (
                        tiles_v[ring, l, pl.ds(rr + 16 * v, 16)])
                wrap_b = bb == s - 1
                block_done = wrap_b & (a == ROWS - 1)

                @pl.when(block_done)
                def _(pm=pm, m=m):
                    store_block(pm, m)
                    @pl.when(m >= 1)
                    def _():
                        wait_store(1 - pm, m - 1)

                bb = lax.select(wrap_b, jnp.int32(0), bb + 1)
                a = lax.select(wrap_b, a + 1, a)
                a = lax.select(block_done, jnp.int32(0), a)
                pm = lax.select(block_done, 1 - pm, pm)
                m = m + block_done.astype(jnp.int32)

            @pl.when(c + NB < nch)
            def _():
                cn = (c + NB) * G
                pltpu.async_copy(
                    table_hbm.at[idx_slice(lax.div(cn, 128),
                                           lax.rem(cn, 128))],
                    tiles_v.at[ring], gsem.at[ring])

            ring = lax.select(ring == NB - 1, jnp.int32(0), ring + 1)
            wrap = o == 128 - G
            o = lax.select(wrap, jnp.int32(0), o + G)
            r = r + wrap.astype(jnp.int32)
            return (ring, r, o, pm, m, a, bb)

        z = jnp.int32(0)
        lax.fori_loop(0, nch, step, (z, z, z, z, z, z, z))

        wait_store((nob - 1) % 2, nob - 1)

    return k


def kernel(tokens, table):
    b, s = tokens.shape
    v = table.shape[0]
    tpw = b * s // NW
    tok3 = tokens.reshape(-1).astype(jnp.int32).reshape(NW, tpw // 128, 128)
    table2 = table.reshape(v // 2, 2 * DIM)
    return _build(b, s)(tok3, table2)


# speedup vs baseline: 1.0787x; 1.0787x over previous
import functools

import jax
import jax.numpy as jnp
from jax import lax
from jax.experimental import pallas as pl
from jax.experimental.pallas import tpu as pltpu
from jax.experimental.pallas import tpu_sc as plsc

DIM = 64
G = 16
NB = 8
ROWS = 8

_info = plsc.get_sparse_core_info()
NC, NS = _info.num_cores, _info.num_subcores
NW = NC * NS


def _build(b, s):
    tpw = b * s // NW
    assert tpw % 128 == 0
    trows = tpw // 128
    tpo = ROWS * s
    assert tpo % G == 0 and tpw % tpo == 0
    cpo = tpo // G
    nch = tpw // G
    mesh = plsc.VectorSubcoreMesh(core_axis_name="c", subcore_axis_name="s")

    @functools.partial(
        pl.kernel,
        mesh=mesh,
        out_type=jax.ShapeDtypeStruct((b, s, DIM), jnp.float32),
        scratch_types=[
            pltpu.VMEM((trows, 128), jnp.int32),
            pltpu.VMEM((trows, 128), jnp.int32),
            pltpu.VMEM((NB, G, 128), jnp.float32),
            pltpu.VMEM((ROWS, s, DIM), jnp.float32),
            pltpu.SemaphoreType.DMA((NB,)),
        ],
        compiler_params=pltpu.CompilerParams(use_tc_tiling_on_sc=True),
    )
    def k(tok_hbm, table_hbm, out_hbm, hvv, tidv, tiles_v, obuf, gsem):
        wid = lax.axis_index("s") * NC + lax.axis_index("c")
        row_base = wid * (tpw // s)
        pltpu.sync_copy(tok_hbm.at[wid], hvv)

        def tid_body(kk, carry):
            r, o = carry
            t16 = hvv[r, pl.ds(o, 16)]
            tidv[r, pl.ds(o, 16)] = lax.shift_right_logical(t16, 1)
            hvv[r, pl.ds(o, 16)] = lax.shift_left(lax.bitwise_and(t16, 1), 6)
            wrap = o == 112
            return (r + wrap.astype(jnp.int32),
                    lax.select(wrap, jnp.int32(0), o + 16))

        lax.fori_loop(0, trows * 8, tid_body, (jnp.int32(0), jnp.int32(0)))

        def idx_slice(r, o):
            return tidv.at[r, pl.ds(o, G)]

        for p in range(NB):
            pltpu.async_copy(
                table_hbm.at[idx_slice((p * G) // 128, (p * G) % 128)],
                tiles_v.at[p], gsem.at[p])

        def step(c, carry):
            ring, r, o, jj, m, a, bb = carry

            pltpu.make_async_copy(table_hbm.at[idx_slice(r, o)],
                                  tiles_v.at[ring], gsem.at[ring]).wait()

            hv = hvv[r, pl.ds(o, 16)]
            for l in range(G):
                rr = hv[l]
                for v in range(DIM // 16):
                    obuf[a, bb, pl.ds(16 * v, 16)] = (
                        tiles_v[ring, l, pl.ds(rr + 16 * v, 16)])
                wrap_b = bb == s - 1
                bb = lax.select(wrap_b, jnp.int32(0), bb + 1)
                a = a + wrap_b.astype(jnp.int32)

            @pl.when(c + NB < nch)
            def _():
                cn = (c + NB) * G
                pltpu.async_copy(
                    table_hbm.at[idx_slice(lax.div(cn, 128),
                                           lax.rem(cn, 128))],
                    tiles_v.at[ring], gsem.at[ring])

            block_done = jj == cpo - 1

            @pl.when(block_done)
            def _():
                pltpu.sync_copy(obuf,
                                out_hbm.at[pl.ds(row_base + m * ROWS, ROWS)])

            ring = lax.select(ring == NB - 1, jnp.int32(0), ring + 1)
            jj = lax.select(block_done, jnp.int32(0), jj + 1)
            m = m + block_done.astype(jnp.int32)
            a = lax.select(block_done, jnp.int32(0), a)
            wrap = o == 128 - G
            o = lax.select(wrap, jnp.int32(0), o + G)
            r = r + wrap.astype(jnp.int32)
            return (ring, r, o, jj, m, a, bb)

        z = jnp.int32(0)
        lax.fori_loop(0, nch, step, (z, z, z, z, z, z, z))

    return k


def kernel(tokens, table):
    b, s = tokens.shape
    v = table.shape[0]
    tpw = b * s // NW
    tok3 = tokens.reshape(-1).astype(jnp.int32).reshape(NW, tpw // 128, 128)
    table2 = table.reshape(v // 2, 2 * DIM)
    return _build(b, s)(tok3, table2)


# speedup vs baseline: 1.1120x; 1.0309x over previous
import functools

import jax
import jax.numpy as jnp
from jax import lax
from jax.experimental import pallas as pl
from jax.experimental.pallas import tpu as pltpu
from jax.experimental.pallas import tpu_sc as plsc

DIM = 64
G = 16
NB = 8
ROWS = 4

_info = plsc.get_sparse_core_info()
NC, NS = _info.num_cores, _info.num_subcores
NW = NC * NS


def _build(b, s):
    tpw = b * s // NW
    assert tpw % 128 == 0
    trows = tpw // 128
    tpo = ROWS * s
    assert tpo % (G // 2) == 0 and tpw % tpo == 0
    nch = tpw // G
    nob = tpw // tpo
    mesh = plsc.VectorSubcoreMesh(core_axis_name="c", subcore_axis_name="s")

    @functools.partial(
        pl.kernel,
        mesh=mesh,
        out_type=jax.ShapeDtypeStruct((b, s, DIM), jnp.float32),
        scratch_types=[
            pltpu.VMEM((trows, 128), jnp.int32),
            pltpu.VMEM((trows, 128), jnp.int32),
            pltpu.VMEM((NB, G, 128), jnp.float32),
            pltpu.VMEM((2, ROWS, s, DIM), jnp.float32),
            pltpu.SemaphoreType.DMA((NB,)),
            pltpu.SemaphoreType.DMA((2,)),
        ],
        compiler_params=pltpu.CompilerParams(use_tc_tiling_on_sc=True),
    )
    def k(tok_hbm, table_hbm, out_hbm, hvv, tidv, tiles_v, obuf, gsem, ssem):
        wid = lax.axis_index("s") * NC + lax.axis_index("c")
        row_base = wid * (tpw // s)
        pltpu.sync_copy(tok_hbm.at[wid], hvv)

        def tid_body(kk, carry):
            r, o = carry
            t16 = hvv[r, pl.ds(o, 16)]
            tidv[r, pl.ds(o, 16)] = lax.shift_right_logical(t16, 1)
            hvv[r, pl.ds(o, 16)] = lax.shift_left(lax.bitwise_and(t16, 1), 6)
            wrap = o == 112
            return (r + wrap.astype(jnp.int32),
                    lax.select(wrap, jnp.int32(0), o + 16))

        lax.fori_loop(0, trows * 8, tid_body, (jnp.int32(0), jnp.int32(0)))

        def idx_slice(r, o):
            return tidv.at[r, pl.ds(o, G)]

        for p in range(NB):
            pltpu.async_copy(
                table_hbm.at[idx_slice((p * G) // 128, (p * G) % 128)],
                tiles_v.at[p], gsem.at[p])

        def store_block(pm, m):
            pltpu.async_copy(obuf.at[pm],
                             out_hbm.at[pl.ds(row_base + m * ROWS, ROWS)],
                             ssem.at[pm])

        def wait_store(pm, m):
            pltpu.make_async_copy(obuf.at[pm],
                                  out_hbm.at[pl.ds(row_base + m * ROWS, ROWS)],
                                  ssem.at[pm]).wait()

        def step(c, carry):
            ring, r, o, pm, m, a, bb = carry

            pltpu.make_async_copy(table_hbm.at[idx_slice(r, o)],
                                  tiles_v.at[ring], gsem.at[ring]).wait()

            hv = hvv[r, pl.ds(o, 16)]
            for h in range(2):
                for l in range(8 * h, 8 * h + 8):
                    rr = hv[l]
                    for v in range(DIM // 16):
                        obuf[pm, a, bb, pl.ds(16 * v, 16)] = (
                            tiles_v[ring, l, pl.ds(rr + 16 * v, 16)])
                    wrap_b = bb == s - 1
                    bb = lax.select(wrap_b, jnp.int32(0), bb + 1)
                    a = a + wrap_b.astype(jnp.int32)

                block_done = a == ROWS

                @pl.when(block_done)
                def _(pm=pm, m=m):
                    store_block(pm, m)
                    @pl.when(m >= 1)
                    def _():
                        wait_store(1 - pm, m - 1)

                a = lax.select(block_done, jnp.int32(0), a)
                pm = lax.select(block_done, 1 - pm, pm)
                m = m + block_done.astype(jnp.int32)

            @pl.when(c + NB < nch)
            def _():
                cn = (c + NB) * G
                pltpu.async_copy(
                    table_hbm.at[idx_slice(lax.div(cn, 128),
                                           lax.rem(cn, 128))],
                    tiles_v.at[ring], gsem.at[ring])

            ring = lax.select(ring == NB - 1, jnp.int32(0), ring + 1)
            wrap = o == 128 - G
            o = lax.select(wrap, jnp.int32(0), o + G)
            r = r + wrap.astype(jnp.int32)
            return (ring, r, o, pm, m, a, bb)

        z = jnp.int32(0)
        lax.fori_loop(0, nch, step, (z, z, z, z, z, z, z))

        wait_store((nob - 1) % 2, nob - 1)

    return k


def kernel(tokens, table):
    b, s = tokens.shape
    v = table.shape[0]
    tpw = b * s // NW
    tok3 = tokens.reshape(-1).astype(jnp.int32).reshape(NW, tpw // 128, 128)
    table2 = table.reshape(v // 2, 2 * DIM)
    return _build(b, s)(tok3, table2)


# speedup vs baseline: 1.1449x; 1.0295x over previous
import functools

import jax
import jax.numpy as jnp
from jax import lax
from jax.experimental import pallas as pl
from jax.experimental.pallas import tpu as pltpu
from jax.experimental.pallas import tpu_sc as plsc

DIM = 64
NB = 8
LAG = 4

_info = plsc.get_sparse_core_info()
NC, NS = _info.num_cores, _info.num_subcores
NW = NC * NS


def _build(b, s):
    rpc = 2
    tpc = rpc * s
    assert tpc <= 128 and b % (NW * rpc) == 0
    nch = b // (NW * rpc)
    assert nch >= NB
    mesh = plsc.VectorSubcoreMesh(core_axis_name="c", subcore_axis_name="s")

    @functools.partial(
        pl.kernel,
        mesh=mesh,
        out_type=jax.ShapeDtypeStruct((b, s, DIM), jnp.float32),
        scratch_types=[
            pltpu.VMEM((nch, tpc), jnp.int32),
            pltpu.VMEM((NB, tpc, DIM), jnp.float32),
            pltpu.SemaphoreType.DMA((NB,)),
            pltpu.SemaphoreType.DMA((NB,)),
        ],
        compiler_params=pltpu.CompilerParams(use_tc_tiling_on_sc=False),
    )
    def k(tok_hbm, table_hbm, out_hbm, idx_v, rows_v, gsem, ssem):
        wid = lax.axis_index("s") * NC + lax.axis_index("c")
        row0 = wid * (nch * rpc)
        pltpu.sync_copy(tok_hbm.at[wid], idx_v)

        def wait_store(c, buf):
            r = row0 + c * rpc
            for q in range(rpc):
                pltpu.make_async_copy(
                    rows_v.at[buf, pl.ds(q * s, s)], out_hbm.at[r + q],
                    ssem.at[buf],
                ).wait()

        def step(i, carry):
            buf = lax.rem(i, NB)

            @pl.when(i < nch)
            def _issue_gather():
                @pl.when(i >= NB)
                def _():
                    wait_store(i - NB, buf)
                pltpu.async_copy(table_hbm.at[idx_v.at[i]], rows_v.at[buf],
                                 gsem.at[buf])

            j = i - LAG

            @pl.when((j >= 0) & (j < nch))
            def _issue_store():
                b2 = lax.rem(j, NB)
                pltpu.make_async_copy(
                    table_hbm.at[idx_v.at[j]], rows_v.at[b2], gsem.at[b2]
                ).wait()
                r = row0 + j * rpc
                for q in range(rpc):
                    pltpu.async_copy(rows_v.at[b2, pl.ds(q * s, s)],
                                     out_hbm.at[r + q], ssem.at[b2])

            return carry

        lax.fori_loop(0, nch + LAG, step, 0)

        def drain(i, carry):
            j = nch - NB + i
            wait_store(j, lax.rem(j, NB))
            return carry

        lax.fori_loop(0, NB, drain, 0)

    return k


def kernel(tokens, table):
    b, s = tokens.shape
    rpc = 2
    nch = b // (NW * rpc)
    tok = tokens.reshape(-1).astype(jnp.int32).reshape(NW, nch, rpc * s)
    return _build(b, s)(tok, table)
